# baseline (speedup 1.0000x reference)
import functools

import jax
import jax.numpy as jnp
from jax import lax
from jax.experimental import pallas as pl
from jax.experimental.pallas import tpu as pltpu
from jax.experimental.pallas import tpu_sc as plsc

VOCAB = 100000
HIDDEN = 128
SEQS = 4096
SLEN = 50
NUM_CORES = 2
NUM_SUBCORES = 16
NW = NUM_CORES * NUM_SUBCORES
APW = SEQS // NW
CCOL = 64
CPP = APW // CCOL
NCHUNK = SLEN * CPP
NBUF = 10
PREF = 5
NGROUPS = NCHUNK // NBUF

_mesh = plsc.VectorSubcoreMesh(core_axis_name="c", subcore_axis_name="s")


@functools.partial(
    pl.kernel,
    mesh=_mesh,
    out_type=(
        jax.ShapeDtypeStruct((SLEN, SEQS, HIDDEN), jnp.float32),
        jax.ShapeDtypeStruct((SLEN, SEQS, HIDDEN), jnp.float32),
    ),
    scratch_types=[pltpu.VMEM((SLEN, APW), jnp.int32)]
    + [pltpu.VMEM((CCOL, HIDDEN), jnp.float32) for _ in range(NBUF)]
    + [pltpu.SemaphoreType.DMA for _ in range(2 * NBUF)],
)
def _emb_gather(ids_hbm, table_hbm, out0_hbm, out1_hbm, idx_v, *bufs):
    rows = bufs[:NBUF]
    gsem = bufs[NBUF:2 * NBUF]
    ssem = bufs[2 * NBUF:]
    outs = (out0_hbm, out1_hbm)
    wid = lax.axis_index("s") * NUM_CORES + lax.axis_index("c")
    base = wid * APW
    pltpu.sync_copy(ids_hbm.at[:, pl.ds(base, APW)], idx_v)

    def gather(c, b):
        s_pos, half = c // CPP, b % CPP
        return pltpu.make_async_copy(
            table_hbm.at[idx_v.at[s_pos, pl.ds(half * CCOL, CCOL)]],
            rows[b], gsem[b])

    def store(c, b, o):
        s_pos, half = c // CPP, b % CPP
        return pltpu.make_async_copy(
            rows[b], outs[o].at[s_pos, pl.ds(base + half * CCOL, CCOL)],
            ssem[b])

    def store_both(c, b):
        store(c, b, 0).start()
        store(c, b, 1).start()

    def wait_stores(b):
        store(0, b, 0).wait()
        store(0, b, 1).wait()

    for c in range(PREF):
        gather(c, c).start()

    def step(c, b, first_round):
        bb = (b + PREF) % NBUF
        if not first_round:
            wait_stores(bb)
        gather(c + PREF, bb).start()
        gather(c, b).wait()
        store_both(c, b)

    for b in range(NBUF):
        step(b, b, first_round=(b + PREF < NBUF))

    def group(g, carry):
        for b in range(NBUF):
            step(g * NBUF + b, b, first_round=False)
        return carry

    lax.fori_loop(1, NGROUPS - 1, group, 0)

    for b in range(NBUF):
        c = (NGROUPS - 1) * NBUF + b
        if c + PREF < NCHUNK:
            bb = (b + PREF) % NBUF
            wait_stores(bb)
            gather(c + PREF, bb).start()
        gather(c, b).wait()
        store_both(c, b)
    for b in range(NBUF):
        wait_stores(b)


def kernel(input_ids, emb_weight):
    ids_t = jnp.transpose(input_ids).astype(jnp.int32)
    o0, o1 = _emb_gather(ids_t, emb_weight)
    h0 = jnp.transpose(o0, (1, 0, 2))
    h1 = jnp.transpose(o1, (1, 0, 2))
    return (h0, h1)


# speedup vs baseline: 1.0132x; 1.0132x over previous
import functools

import jax
import jax.numpy as jnp
from jax import lax
from jax.experimental import pallas as pl
from jax.experimental.pallas import tpu as pltpu
from jax.experimental.pallas import tpu_sc as plsc

VOCAB = 100000
HIDDEN = 128
SEQS = 4096
SLEN = 50
NUM_CORES = 2
NUM_SUBCORES = 16
NW = NUM_CORES * NUM_SUBCORES
APW = SEQS // NW
NCHUNK = SLEN
NBUF = 5
PREF = 2
NGROUPS = NCHUNK // NBUF

_mesh = plsc.VectorSubcoreMesh(core_axis_name="c", subcore_axis_name="s")


@functools.partial(
    pl.kernel,
    mesh=_mesh,
    out_type=(
        jax.ShapeDtypeStruct((SLEN, SEQS, HIDDEN), jnp.float32),
        jax.ShapeDtypeStruct((SLEN, SEQS, HIDDEN), jnp.float32),
    ),
    scratch_types=[pltpu.VMEM((NCHUNK, APW), jnp.int32)]
    + [pltpu.VMEM((APW, HIDDEN), jnp.float32) for _ in range(NBUF)]
    + [pltpu.SemaphoreType.DMA for _ in range(2 * NBUF)],
)
def _emb_gather(ids_hbm, table_hbm, out0_hbm, out1_hbm, idx_v, *bufs):
    rows = bufs[:NBUF]
    gsem = bufs[NBUF:2 * NBUF]
    ssem = bufs[2 * NBUF:]
    outs = (out0_hbm, out1_hbm)
    wid = lax.axis_index("s") * NUM_CORES + lax.axis_index("c")
    base = wid * APW
    pltpu.sync_copy(ids_hbm.at[:, pl.ds(base, APW)], idx_v)

    def gather(c, b):
        return pltpu.make_async_copy(
            table_hbm.at[idx_v.at[c]], rows[b], gsem[b])

    def store(c, b, o):
        return pltpu.make_async_copy(
            rows[b], outs[o].at[c, pl.ds(base, APW)], ssem[b])

    def store_both(c, b):
        store(c, b, 0).start()
        store(c, b, 1).start()

    def wait_stores(b):
        store(0, b, 0).wait()
        store(0, b, 1).wait()

    for c in range(PREF):
        gather(c, c).start()

    def step(c, b, first_round):
        bb = (b + PREF) % NBUF
        if not first_round:
            wait_stores(bb)
        gather(c + PREF, bb).start()
        gather(c, b).wait()
        store_both(c, b)

    for b in range(NBUF):
        step(b, b, first_round=(b + PREF < NBUF))

    def group(g, carry):
        for b in range(NBUF):
            step(g * NBUF + b, b, first_round=False)
        return carry

    lax.fori_loop(1, NGROUPS - 1, group, 0)

    for b in range(NBUF):
        c = (NGROUPS - 1) * NBUF + b
        if c + PREF < NCHUNK:
            bb = (b + PREF) % NBUF
            wait_stores(bb)
            gather(c + PREF, bb).start()
        gather(c, b).wait()
        store_both(c, b)
    for b in range(NBUF):
        wait_stores(b)


def kernel(input_ids, emb_weight):
    ids_t = jnp.transpose(input_ids).astype(jnp.int32)
    o0, o1 = _emb_gather(ids_t, emb_weight)
    h0 = jnp.transpose(o0, (1, 0, 2))
    h1 = jnp.transpose(o1, (1, 0, 2))
    return (h0, h1)
